# Initial kernel scaffold; baseline (speedup 1.0000x reference)
import jax
import jax.numpy as jnp
from jax.experimental import pallas as pl


def kernel(x, w, b):
    raise NotImplementedError("write your pallas kernel here")


# speedup vs baseline: 3.0636x; 3.0636x over previous
import jax
import jax.numpy as jnp
from jax.experimental import pallas as pl
from jax.experimental.pallas import tpu as pltpu

_DOT_DIMS = (((1,), (0,)), ((), ()))


def _mm_bias_kernel(x_ref, w_ref, b_ref, o_ref):
    acc = jax.lax.dot_general(x_ref[...], w_ref[...],
                              dimension_numbers=_DOT_DIMS,
                              preferred_element_type=jnp.float32)
    o_ref[...] = (acc + b_ref[...].astype(jnp.float32)).astype(o_ref.dtype)


def _round_up(v, m):
    return ((v + m - 1) // m) * m


def kernel(x, w, b):
    B, K = x.shape
    K2, N = w.shape
    assert K == K2, (K, K2)

    xb = x.astype(jnp.bfloat16)
    wb = w.astype(jnp.bfloat16)

    tm = min(1024, _round_up(B, 256))
    tn = min(1024, _round_up(N, 256))
    Mp, Np = _round_up(B, tm), _round_up(N, tn)
    if Mp != B:
        xb = jnp.pad(xb, ((0, Mp - B), (0, 0)))
    if Np != N:
        wb = jnp.pad(wb, ((0, 0), (0, Np - N)))
        b = jnp.pad(b, ((0, 0), (0, Np - N)))

    grid = (Mp // tm, Np // tn)
    out = pl.pallas_call(
        _mm_bias_kernel,
        out_shape=jax.ShapeDtypeStruct((Mp, Np), x.dtype),
        grid=grid,
        in_specs=[
            pl.BlockSpec((tm, K), lambda i, j: (i, 0)),
            pl.BlockSpec((K, tn), lambda i, j: (0, j)),
            pl.BlockSpec((1, tn), lambda i, j: (0, j)),
        ],
        out_specs=pl.BlockSpec((tm, tn), lambda i, j: (i, j)),
        compiler_params=pltpu.CompilerParams(
            dimension_semantics=("parallel", "parallel"),
            vmem_limit_bytes=60 << 20,
        ),
    )(xb, wb, b)

    return out[:B, :N] if (Mp, Np) != (B, N) else out


# speedup vs baseline: 3.5780x; 1.1679x over previous
import jax
import jax.numpy as jnp
from jax.experimental import pallas as pl
from jax.experimental.pallas import tpu as pltpu

_DOT_DIMS = (((1,), (0,)), ((), ()))


def _mm_bias_kernel(x_ref, w_ref, b_ref, o_ref):
    xb = x_ref[...].astype(jnp.bfloat16)
    acc = jax.lax.dot_general(xb, w_ref[...],
                              dimension_numbers=_DOT_DIMS,
                              preferred_element_type=jnp.float32)
    o_ref[...] = (acc + b_ref[...].astype(jnp.float32)).astype(o_ref.dtype)


def _round_up(v, m):
    return ((v + m - 1) // m) * m


def kernel(x, w, b):
    B, K = x.shape
    K2, N = w.shape
    assert K == K2, (K, K2)

    wb = w.astype(jnp.bfloat16)

    tm = min(256, _round_up(B, 8))
    Mp = _round_up(B, tm)
    if Mp != B:
        x = jnp.pad(x, ((0, Mp - B), (0, 0)))

    out = pl.pallas_call(
        _mm_bias_kernel,
        out_shape=jax.ShapeDtypeStruct((Mp, N), x.dtype),
        grid=(Mp // tm,),
        in_specs=[
            pl.BlockSpec((tm, K), lambda i: (i, 0)),
            pl.BlockSpec((K, N), lambda i: (0, 0)),
            pl.BlockSpec((1, N), lambda i: (0, 0)),
        ],
        out_specs=pl.BlockSpec((tm, N), lambda i: (i, 0)),
        compiler_params=pltpu.CompilerParams(
            dimension_semantics=("parallel",),
            vmem_limit_bytes=60 << 20,
        ),
    )(x, wb, b)

    return out[:B] if Mp != B else out
